# baseline (speedup 1.0000x reference)
import functools
import jax
import jax.numpy as jnp
from jax import lax
from jax.experimental import pallas as pl
from jax.experimental.pallas import tpu as pltpu
from jax.experimental.pallas import tpu_sc as plsc

E = 8
D = 1024
K = 2
N = 4096
B = 128
P = N * K
LMAX = P + E * B
NB = LMAX // B
CHUNK = 128
NCH = N // CHUNK

NC = 2
NS = 16
NW = NC * NS
TPW = N // NW
CH = 32
NCHW = TPW // CH
CC = 16
NCC = TPW // CC


def _router_body(x_ref, gw_ref, gb_ref, temp_ref, p0_ref, p1_ref, be_ref):
    scores = lax.dot_general(x_ref[...], gw_ref[...], (((1,), (1,)), ((), ())),
                             preferred_element_type=jnp.float32)
    scores = (scores + gb_ref[...]) / temp_ref[0, 0]
    iota = lax.broadcasted_iota(jnp.int32, (N, E), 1)
    vmax1 = jnp.max(scores, axis=1, keepdims=True)
    a1 = jnp.min(jnp.where(scores == vmax1, iota, E), axis=1, keepdims=True)
    s2 = jnp.where(iota == a1, jnp.float32(-jnp.inf), scores)
    vmax2 = jnp.max(s2, axis=1, keepdims=True)
    a2 = jnp.min(jnp.where(s2 == vmax2, iota, E), axis=1, keepdims=True)
    oh0 = (iota == a1).astype(jnp.float32)
    oh1 = (iota == a2).astype(jnp.float32)
    ohs = oh0 + oh1

    oh3 = ohs.reshape(NCH, CHUNK, E)
    ri = lax.broadcasted_iota(jnp.int32, (CHUNK, CHUNK), 0)
    ci = lax.broadcasted_iota(jnp.int32, (CHUNK, CHUNK), 1)
    lstrict = (ci < ri).astype(jnp.float32)
    lb = jnp.broadcast_to(lstrict[None], (NCH, CHUNK, CHUNK))
    within = lax.dot_general(lb, oh3, (((2,), (1,)), ((0,), (0,))),
                             preferred_element_type=jnp.float32)
    ctot = jnp.sum(oh3, axis=1)
    r32 = lax.broadcasted_iota(jnp.int32, (NCH, NCH), 0)
    c32 = lax.broadcasted_iota(jnp.int32, (NCH, NCH), 1)
    l32 = (c32 < r32).astype(jnp.float32)
    coff = lax.dot_general(l32, ctot, (((1,), (0,)), ((), ())),
                           preferred_element_type=jnp.float32)
    cs = (within + coff[:, None, :]).reshape(N, E)

    counts = jnp.sum(ctot, axis=0, keepdims=True).astype(jnp.int32)
    pc = ((counts + (B - 1)) // B) * B
    er = lax.broadcasted_iota(jnp.int32, (E, E), 0)
    ec = lax.broadcasted_iota(jnp.int32, (E, E), 1)
    u8 = (er < ec).astype(jnp.float32)
    ss = lax.dot_general(pc.astype(jnp.float32), u8, (((1,), (0,)), ((), ())),
                         preferred_element_type=jnp.float32)

    p0_ref[...] = jnp.sum(oh0 * (cs + ss), axis=1, keepdims=True).astype(jnp.int32)
    p1_ref[...] = jnp.sum(oh1 * (cs + ss), axis=1, keepdims=True).astype(jnp.int32)

    bb = lax.broadcasted_iota(jnp.int32, (NB, E), 0) * B
    be = jnp.sum((bb >= ss.astype(jnp.int32)).astype(jnp.int32), axis=1,
                 keepdims=True) - 1
    be_ref[...] = be


def _router(x, gate_W, gate_b, temperature):
    return pl.pallas_call(
        _router_body,
        in_specs=[
            pl.BlockSpec((N, D), lambda: (0, 0)),
            pl.BlockSpec((E, D), lambda: (0, 0)),
            pl.BlockSpec((1, E), lambda: (0, 0)),
            pl.BlockSpec((1, 1), lambda: (0, 0)),
        ],
        out_specs=[
            pl.BlockSpec((N, 1), lambda: (0, 0)),
            pl.BlockSpec((N, 1), lambda: (0, 0)),
            pl.BlockSpec((NB, 1), lambda: (0, 0)),
        ],
        out_shape=[
            jax.ShapeDtypeStruct((N, 1), jnp.int32),
            jax.ShapeDtypeStruct((N, 1), jnp.int32),
            jax.ShapeDtypeStruct((NB, 1), jnp.int32),
        ],
    )(x, gate_W, gate_b.reshape(1, E), temperature.reshape(1, 1))


def _scatter_sc(x, p0, p1):
    mesh = plsc.VectorSubcoreMesh(core_axis_name="c", subcore_axis_name="s")

    @functools.partial(
        pl.kernel,
        out_type=jax.ShapeDtypeStruct((LMAX, 8, 128), jnp.float32),
        mesh=mesh,
        scratch_types=[
            pltpu.VMEM((2 * NCHW, CH), jnp.int32),
            pltpu.VMEM((CH, 8, 128), jnp.float32),
            pltpu.VMEM((CH, 8, 128), jnp.float32),
            pltpu.SemaphoreType.DMA,
            pltpu.SemaphoreType.DMA,
        ],
    )
    def scatter_k(x_hbm, p0_hbm, p1_hbm, xs_hbm, idx_v, xb0, xb1, sem0, sem1):
        wid = lax.axis_index("s") * NC + lax.axis_index("c")
        base = wid * TPW
        for j in range(NCHW):
            pltpu.sync_copy(p0_hbm.at[pl.ds(base + j * CH, CH)], idx_v.at[j])
            pltpu.sync_copy(p1_hbm.at[pl.ds(base + j * CH, CH)],
                            idx_v.at[NCHW + j])
        bufs = (xb0, xb1)
        sems = (sem0, sem1)
        pend = [None] * NCHW
        for j in range(NCHW):
            if j >= 2:
                for c in pend[j - 2]:
                    c.wait()
            buf = bufs[j % 2]
            sem = sems[j % 2]
            pltpu.sync_copy(x_hbm.at[pl.ds(base + j * CH, CH)], buf)
            c0 = pltpu.async_copy(buf, xs_hbm.at[idx_v.at[j]], sem)
            c1 = pltpu.async_copy(buf, xs_hbm.at[idx_v.at[NCHW + j]], sem)
            pend[j] = (c0, c1)
        for j in range(max(0, NCHW - 2), NCHW):
            for c in pend[j]:
                c.wait()

    return scatter_k(x, p0, p1)


def _mm_body(be_ref, xs_ref, w_ref, b_ref, y_ref):
    y = lax.dot_general(
        xs_ref[...].reshape(B, D), w_ref[0].reshape(D, D),
        (((1,), (0,)), ((), ())),
        preferred_element_type=jnp.float32) + b_ref[0]
    y_ref[...] = y.reshape(B, 8, 128)


def _matmul(be, xs, expert_W, expert_b):
    grid_spec = pltpu.PrefetchScalarGridSpec(
        num_scalar_prefetch=1,
        grid=(NB,),
        in_specs=[
            pl.BlockSpec((B, 8, 128), lambda b, ids: (b, 0, 0)),
            pl.BlockSpec((1, 8, 128, D), lambda b, ids: (ids[b], 0, 0, 0)),
            pl.BlockSpec((1, 1, D), lambda b, ids: (ids[b], 0, 0)),
        ],
        out_specs=pl.BlockSpec((B, 8, 128), lambda b, ids: (b, 0, 0)),
    )
    return pl.pallas_call(
        _mm_body,
        grid_spec=grid_spec,
        out_shape=jax.ShapeDtypeStruct((LMAX, 8, 128), jnp.float32),
    )(be, xs, expert_W.reshape(E, 8, 128, D), expert_b.reshape(E, 1, D))


def _combine_sc(y, p0, p1):
    mesh = plsc.VectorSubcoreMesh(core_axis_name="c", subcore_axis_name="s")

    @functools.partial(
        pl.kernel,
        out_type=jax.ShapeDtypeStruct((N, 8, 128), jnp.float32),
        mesh=mesh,
        scratch_types=[
            pltpu.VMEM((2 * NCC, CC), jnp.int32),
            pltpu.VMEM((CC, 8, 128), jnp.float32),
            pltpu.VMEM((CC, 8, 128), jnp.float32),
            pltpu.VMEM((CC, 8, 128), jnp.float32),
            pltpu.VMEM((CC, 8, 128), jnp.float32),
            pltpu.SemaphoreType.DMA,
            pltpu.SemaphoreType.DMA,
        ],
    )
    def combine_k(y_hbm, p0_hbm, p1_hbm, out_hbm, idx_v,
                  a0, a1, b0, b1, sem0, sem1):
        wid = lax.axis_index("s") * NC + lax.axis_index("c")
        base = wid * TPW
        for j in range(NCC):
            pltpu.sync_copy(p0_hbm.at[pl.ds(base + j * CC, CC)], idx_v.at[j])
            pltpu.sync_copy(p1_hbm.at[pl.ds(base + j * CC, CC)],
                            idx_v.at[NCC + j])
        g0 = (a0, b0)
        g1 = (a1, b1)
        sems = (sem0, sem1)
        pend = [None] * NCC
        pend[0] = (pltpu.async_copy(y_hbm.at[idx_v.at[0]], g0[0], sem0),
                   pltpu.async_copy(y_hbm.at[idx_v.at[NCC]], g1[0], sem0))
        for j in range(NCC):
            if j + 1 < NCC:
                p = (j + 1) % 2
                pend[j + 1] = (
                    pltpu.async_copy(y_hbm.at[idx_v.at[j + 1]], g0[p], sems[p]),
                    pltpu.async_copy(y_hbm.at[idx_v.at[NCC + j + 1]], g1[p],
                                     sems[p]))
            for c in pend[j]:
                c.wait()
            u = g0[j % 2]
            v = g1[j % 2]

            def row_add(r, _):
                for c in range(8):
                    for q in range(8):
                        sl = pl.ds(q * 16, 16)
                        u[r, c, sl] = u[r, c, sl] + v[r, c, sl]
                return 0

            lax.fori_loop(0, CC, row_add, 0)
            pltpu.sync_copy(u, out_hbm.at[pl.ds(base + j * CC, CC)])

    return combine_k(y, p0, p1)


def kernel(x, gate_W, gate_b, temperature, expert_W, expert_b):
    p0, p1, be = _router(x, gate_W, gate_b, temperature)
    p0 = p0.reshape(N)
    p1 = p1.reshape(N)
    be = be.reshape(NB)
    xs = _scatter_sc(x.reshape(N, 8, 128), p0, p1)
    y = _matmul(be, xs, expert_W, expert_b)
    out = _combine_sc(y, p0, p1)
    return out.reshape(N, 1, D)


# speedup vs baseline: 1.1268x; 1.1268x over previous
import functools
import jax
import jax.numpy as jnp
from jax import lax
from jax.experimental import pallas as pl
from jax.experimental.pallas import tpu as pltpu
from jax.experimental.pallas import tpu_sc as plsc

E = 8
D = 1024
K = 2
N = 4096
B = 512
P = N * K
LMAX = P + E * B
NB = LMAX // B
CHUNK = 128
NCH = N // CHUNK

NC = 2
NS = 16
NW = NC * NS
TPW = N // NW
CH = 32
NCHW = TPW // CH
CC = 16
NCC = TPW // CC


def _router_body(x_ref, gw_ref, gb_ref, temp_ref, p0_ref, p1_ref, be_ref):
    scores = lax.dot_general(x_ref[...], gw_ref[...], (((1,), (1,)), ((), ())),
                             preferred_element_type=jnp.float32)
    scores = (scores + gb_ref[...]) / temp_ref[0, 0]
    iota = lax.broadcasted_iota(jnp.int32, (N, E), 1)
    vmax1 = jnp.max(scores, axis=1, keepdims=True)
    a1 = jnp.min(jnp.where(scores == vmax1, iota, E), axis=1, keepdims=True)
    s2 = jnp.where(iota == a1, jnp.float32(-jnp.inf), scores)
    vmax2 = jnp.max(s2, axis=1, keepdims=True)
    a2 = jnp.min(jnp.where(s2 == vmax2, iota, E), axis=1, keepdims=True)
    oh0 = (iota == a1).astype(jnp.float32)
    oh1 = (iota == a2).astype(jnp.float32)
    ohs = oh0 + oh1

    oh3 = ohs.reshape(NCH, CHUNK, E)
    ri = lax.broadcasted_iota(jnp.int32, (CHUNK, CHUNK), 0)
    ci = lax.broadcasted_iota(jnp.int32, (CHUNK, CHUNK), 1)
    lstrict = (ci < ri).astype(jnp.float32)
    lb = jnp.broadcast_to(lstrict[None], (NCH, CHUNK, CHUNK))
    within = lax.dot_general(lb, oh3, (((2,), (1,)), ((0,), (0,))),
                             preferred_element_type=jnp.float32)
    ctot = jnp.sum(oh3, axis=1)
    r32 = lax.broadcasted_iota(jnp.int32, (NCH, NCH), 0)
    c32 = lax.broadcasted_iota(jnp.int32, (NCH, NCH), 1)
    l32 = (c32 < r32).astype(jnp.float32)
    coff = lax.dot_general(l32, ctot, (((1,), (0,)), ((), ())),
                           preferred_element_type=jnp.float32)
    cs = (within + coff[:, None, :]).reshape(N, E)

    counts = jnp.sum(ctot, axis=0, keepdims=True).astype(jnp.int32)
    pc = ((counts + (B - 1)) // B) * B
    er = lax.broadcasted_iota(jnp.int32, (E, E), 0)
    ec = lax.broadcasted_iota(jnp.int32, (E, E), 1)
    u8 = (er < ec).astype(jnp.float32)
    ss = lax.dot_general(pc.astype(jnp.float32), u8, (((1,), (0,)), ((), ())),
                         preferred_element_type=jnp.float32)

    p0_ref[...] = jnp.sum(oh0 * (cs + ss), axis=1, keepdims=True).astype(jnp.int32)
    p1_ref[...] = jnp.sum(oh1 * (cs + ss), axis=1, keepdims=True).astype(jnp.int32)

    bb = lax.broadcasted_iota(jnp.int32, (NB, E), 0) * B
    be = jnp.sum((bb >= ss.astype(jnp.int32)).astype(jnp.int32), axis=1,
                 keepdims=True) - 1
    be_ref[...] = be


def _router(x, gate_W, gate_b, temperature):
    return pl.pallas_call(
        _router_body,
        in_specs=[
            pl.BlockSpec((N, D), lambda: (0, 0)),
            pl.BlockSpec((E, D), lambda: (0, 0)),
            pl.BlockSpec((1, E), lambda: (0, 0)),
            pl.BlockSpec((1, 1), lambda: (0, 0)),
        ],
        out_specs=[
            pl.BlockSpec((N, 1), lambda: (0, 0)),
            pl.BlockSpec((N, 1), lambda: (0, 0)),
            pl.BlockSpec((NB, 1), lambda: (0, 0)),
        ],
        out_shape=[
            jax.ShapeDtypeStruct((N, 1), jnp.int32),
            jax.ShapeDtypeStruct((N, 1), jnp.int32),
            jax.ShapeDtypeStruct((NB, 1), jnp.int32),
        ],
    )(x, gate_W, gate_b.reshape(1, E), temperature.reshape(1, 1))


def _scatter_sc(x, p0, p1):
    mesh = plsc.VectorSubcoreMesh(core_axis_name="c", subcore_axis_name="s")

    @functools.partial(
        pl.kernel,
        out_type=jax.ShapeDtypeStruct((LMAX, 8, 128), jnp.float32),
        mesh=mesh,
        scratch_types=[
            pltpu.VMEM((2 * NCHW, CH), jnp.int32),
            pltpu.VMEM((CH, 8, 128), jnp.float32),
            pltpu.VMEM((CH, 8, 128), jnp.float32),
            pltpu.SemaphoreType.DMA,
            pltpu.SemaphoreType.DMA,
        ],
    )
    def scatter_k(x_hbm, p0_hbm, p1_hbm, xs_hbm, idx_v, xb0, xb1, sem0, sem1):
        wid = lax.axis_index("s") * NC + lax.axis_index("c")
        base = wid * TPW
        for j in range(NCHW):
            pltpu.sync_copy(p0_hbm.at[pl.ds(base + j * CH, CH)], idx_v.at[j])
            pltpu.sync_copy(p1_hbm.at[pl.ds(base + j * CH, CH)],
                            idx_v.at[NCHW + j])
        bufs = (xb0, xb1)
        sems = (sem0, sem1)
        pend = [None] * NCHW
        for j in range(NCHW):
            if j >= 2:
                for c in pend[j - 2]:
                    c.wait()
            buf = bufs[j % 2]
            sem = sems[j % 2]
            pltpu.sync_copy(x_hbm.at[pl.ds(base + j * CH, CH)], buf)
            c0 = pltpu.async_copy(buf, xs_hbm.at[idx_v.at[j]], sem)
            c1 = pltpu.async_copy(buf, xs_hbm.at[idx_v.at[NCHW + j]], sem)
            pend[j] = (c0, c1)
        for j in range(max(0, NCHW - 2), NCHW):
            for c in pend[j]:
                c.wait()

    return scatter_k(x, p0, p1)


def _mm_body(be_ref, xs_ref, w_ref, b_ref, y_ref):
    y = lax.dot_general(
        xs_ref[...].reshape(B, D), w_ref[0].reshape(D, D),
        (((1,), (0,)), ((), ())),
        preferred_element_type=jnp.float32) + b_ref[0]
    y_ref[...] = y.reshape(B, 8, 128)


def _matmul(be, xs, expert_W, expert_b):
    grid_spec = pltpu.PrefetchScalarGridSpec(
        num_scalar_prefetch=1,
        grid=(NB,),
        in_specs=[
            pl.BlockSpec((B, 8, 128), lambda b, ids: (b, 0, 0)),
            pl.BlockSpec((1, 8, 128, D), lambda b, ids: (ids[b], 0, 0, 0)),
            pl.BlockSpec((1, 1, D), lambda b, ids: (ids[b], 0, 0)),
        ],
        out_specs=pl.BlockSpec((B, 8, 128), lambda b, ids: (b, 0, 0)),
    )
    return pl.pallas_call(
        _mm_body,
        grid_spec=grid_spec,
        out_shape=jax.ShapeDtypeStruct((LMAX, 8, 128), jnp.float32),
    )(be, xs, expert_W.reshape(E, 8, 128, D), expert_b.reshape(E, 1, D))


def _combine_sc(y, p0, p1):
    mesh = plsc.VectorSubcoreMesh(core_axis_name="c", subcore_axis_name="s")

    @functools.partial(
        pl.kernel,
        out_type=jax.ShapeDtypeStruct((N, 8, 128), jnp.float32),
        mesh=mesh,
        scratch_types=[
            pltpu.VMEM((2 * NCC, CC), jnp.int32),
            pltpu.VMEM((CC, 8, 128), jnp.float32),
            pltpu.VMEM((CC, 8, 128), jnp.float32),
            pltpu.VMEM((CC, 8, 128), jnp.float32),
            pltpu.VMEM((CC, 8, 128), jnp.float32),
            pltpu.SemaphoreType.DMA,
            pltpu.SemaphoreType.DMA,
        ],
    )
    def combine_k(y_hbm, p0_hbm, p1_hbm, out_hbm, idx_v,
                  a0, a1, b0, b1, sem0, sem1):
        wid = lax.axis_index("s") * NC + lax.axis_index("c")
        base = wid * TPW
        for j in range(NCC):
            pltpu.sync_copy(p0_hbm.at[pl.ds(base + j * CC, CC)], idx_v.at[j])
            pltpu.sync_copy(p1_hbm.at[pl.ds(base + j * CC, CC)],
                            idx_v.at[NCC + j])
        g0 = (a0, b0)
        g1 = (a1, b1)
        sems = (sem0, sem1)
        pend = [None] * NCC
        pend[0] = (pltpu.async_copy(y_hbm.at[idx_v.at[0]], g0[0], sem0),
                   pltpu.async_copy(y_hbm.at[idx_v.at[NCC]], g1[0], sem0))
        for j in range(NCC):
            if j + 1 < NCC:
                p = (j + 1) % 2
                pend[j + 1] = (
                    pltpu.async_copy(y_hbm.at[idx_v.at[j + 1]], g0[p], sems[p]),
                    pltpu.async_copy(y_hbm.at[idx_v.at[NCC + j + 1]], g1[p],
                                     sems[p]))
            for c in pend[j]:
                c.wait()
            u = g0[j % 2]
            v = g1[j % 2]

            def row_add(r, _):
                for c in range(8):
                    for q in range(8):
                        sl = pl.ds(q * 16, 16)
                        u[r, c, sl] = u[r, c, sl] + v[r, c, sl]
                return 0

            lax.fori_loop(0, CC, row_add, 0)
            pltpu.sync_copy(u, out_hbm.at[pl.ds(base + j * CC, CC)])

    return combine_k(y, p0, p1)


def kernel(x, gate_W, gate_b, temperature, expert_W, expert_b):
    p0, p1, be = _router(x, gate_W, gate_b, temperature)
    p0 = p0.reshape(N)
    p1 = p1.reshape(N)
    be = be.reshape(NB)
    xs = _scatter_sc(x.reshape(N, 8, 128), p0, p1)
    y = _matmul(be, xs, expert_W, expert_b)
    out = _combine_sc(y, p0, p1)
    return out.reshape(N, 1, D)
